# baseline (speedup 1.0000x reference)
import functools

import jax
import jax.numpy as jnp
from jax.experimental import pallas as pl
from jax.experimental.pallas import tpu as pltpu

TOP_K = 2
NUM_EXPERTS = 8
D_MODEL = 1024
TOKENS = 4096
TBLK = 512


def _moe_block(x_ref, xb_ref, wg_ref, bg_ref, we_ref, be_ref, out_ref,
               probs_ref, gate_ref):
    e = pl.program_id(1)

    @pl.when(e == 0)
    def _router():
        x = x_ref[...]
        scores = jnp.dot(x, wg_ref[...], preferred_element_type=jnp.float32)
        scores = scores + bg_ref[...]
        idx = jax.lax.broadcasted_iota(jnp.int32, scores.shape, 1)
        m1 = jnp.max(scores, axis=1, keepdims=True)
        i1 = jnp.min(jnp.where(scores == m1, idx, NUM_EXPERTS), axis=1,
                     keepdims=True)
        masked = jnp.where(idx == i1, -jnp.inf, scores)
        m2 = jnp.max(masked, axis=1, keepdims=True)
        i2 = jnp.min(jnp.where(masked == m2, idx, NUM_EXPERTS), axis=1,
                     keepdims=True)
        e2 = jnp.exp(m2 - m1)
        denom = 1.0 + e2
        p0 = 1.0 / denom
        p1 = e2 / denom
        probs_ref[...] = jnp.concatenate([p0, p1], axis=1)
        gate_ref[...] = jnp.where(idx == i1, p0, 0.0) + jnp.where(
            idx == i2, p1, 0.0)
        out_ref[...] = jnp.zeros_like(out_ref)

    gate = gate_ref[...]
    eidx = jax.lax.broadcasted_iota(jnp.int32, gate.shape, 1)
    gate_col = jnp.sum(jnp.where(eidx == e, gate, 0.0), axis=1, keepdims=True)
    y = jnp.dot(xb_ref[...], we_ref[0], preferred_element_type=jnp.float32)
    out_ref[...] += gate_col * (y + be_ref[0])


def kernel(inputs, Wg, bg, We, be):
    n_tb = TOKENS // TBLK
    grid = (n_tb, NUM_EXPERTS)
    out, probs = pl.pallas_call(
        _moe_block,
        grid=grid,
        in_specs=[
            pl.BlockSpec((TBLK, D_MODEL), lambda t, e: (t, 0)),
            pl.BlockSpec((TBLK, D_MODEL), lambda t, e: (t, 0)),
            pl.BlockSpec((D_MODEL, NUM_EXPERTS), lambda t, e: (0, 0)),
            pl.BlockSpec((1, NUM_EXPERTS), lambda t, e: (0, 0)),
            pl.BlockSpec((1, D_MODEL, D_MODEL), lambda t, e: (e, 0, 0)),
            pl.BlockSpec((1, 1, D_MODEL), lambda t, e: (e, 0, 0)),
        ],
        out_specs=[
            pl.BlockSpec((TBLK, D_MODEL), lambda t, e: (t, 0)),
            pl.BlockSpec((TBLK, TOP_K), lambda t, e: (t, 0)),
        ],
        out_shape=[
            jax.ShapeDtypeStruct((TOKENS, D_MODEL), jnp.float32),
            jax.ShapeDtypeStruct((TOKENS, TOP_K), jnp.float32),
        ],
        scratch_shapes=[pltpu.VMEM((TBLK, NUM_EXPERTS), jnp.float32)],
    )(inputs, inputs.astype(jnp.bfloat16), Wg, bg.reshape(1, NUM_EXPERTS),
      We.astype(jnp.bfloat16), be.reshape(NUM_EXPERTS, 1, D_MODEL))
    return (out, probs)


# speedup vs baseline: 1.2518x; 1.2518x over previous
import functools

import jax
import jax.numpy as jnp
from jax.experimental import pallas as pl
from jax.experimental.pallas import tpu as pltpu

TOP_K = 2
NUM_EXPERTS = 8
D_MODEL = 1024
TOKENS = 4096
TBLK = 512


def _moe_block(x_ref, xb_ref, wg_ref, bg_ref, we_ref, be_ref, out_ref,
               probs_ref):
    x = x_ref[...]
    scores = jnp.dot(x, wg_ref[...], preferred_element_type=jnp.float32)
    scores = scores + bg_ref[...]
    idx = jax.lax.broadcasted_iota(jnp.int32, scores.shape, 1)
    m1 = jnp.max(scores, axis=1, keepdims=True)
    i1 = jnp.min(jnp.where(scores == m1, idx, NUM_EXPERTS), axis=1,
                 keepdims=True)
    masked = jnp.where(idx == i1, -jnp.inf, scores)
    m2 = jnp.max(masked, axis=1, keepdims=True)
    i2 = jnp.min(jnp.where(masked == m2, idx, NUM_EXPERTS), axis=1,
                 keepdims=True)
    e2 = jnp.exp(m2 - m1)
    denom = 1.0 + e2
    p0 = 1.0 / denom
    p1 = e2 / denom
    probs_ref[...] = jnp.concatenate([p0, p1], axis=1)
    gate = jnp.where(idx == i1, p0, 0.0) + jnp.where(idx == i2, p1, 0.0)

    xb = xb_ref[...]
    acc = jnp.zeros((TBLK, D_MODEL), jnp.float32)
    for e in range(NUM_EXPERTS):
        y = jnp.dot(xb, we_ref[e], preferred_element_type=jnp.float32)
        acc += gate[:, e:e + 1] * (y + be_ref[e])
    out_ref[...] = acc


def kernel(inputs, Wg, bg, We, be):
    n_tb = TOKENS // TBLK
    out, probs = pl.pallas_call(
        _moe_block,
        grid=(n_tb,),
        in_specs=[
            pl.BlockSpec((TBLK, D_MODEL), lambda t: (t, 0)),
            pl.BlockSpec((TBLK, D_MODEL), lambda t: (t, 0)),
            pl.BlockSpec((D_MODEL, NUM_EXPERTS), lambda t: (0, 0)),
            pl.BlockSpec((1, NUM_EXPERTS), lambda t: (0, 0)),
            pl.BlockSpec((NUM_EXPERTS, D_MODEL, D_MODEL), lambda t: (0, 0, 0)),
            pl.BlockSpec((NUM_EXPERTS, 1, D_MODEL), lambda t: (0, 0, 0)),
        ],
        out_specs=[
            pl.BlockSpec((TBLK, D_MODEL), lambda t: (t, 0)),
            pl.BlockSpec((TBLK, TOP_K), lambda t: (t, 0)),
        ],
        out_shape=[
            jax.ShapeDtypeStruct((TOKENS, D_MODEL), jnp.float32),
            jax.ShapeDtypeStruct((TOKENS, TOP_K), jnp.float32),
        ],
    )(inputs, inputs.astype(jnp.bfloat16), Wg, bg.reshape(1, NUM_EXPERTS),
      We.astype(jnp.bfloat16), be.reshape(NUM_EXPERTS, 1, D_MODEL))
    return (out, probs)
